# Initial kernel scaffold; baseline (speedup 1.0000x reference)
import jax
import jax.numpy as jnp
from jax.experimental import pallas as pl


def kernel(x_author, x_paper, edge_src, edge_dst, W1w, b1w, W1b, b1b, W2w, b2w, W2b, b2b, W3w, b3w, W3b, b3b):
    raise NotImplementedError("write your pallas kernel here")


# speedup vs baseline: 4.6062x; 4.6062x over previous
import functools

import jax
import jax.numpy as jnp
from jax import lax
from jax.experimental import pallas as pl
from jax.experimental.pallas import tpu as pltpu
from jax.experimental.pallas import tpu_sc as plsc

N_A = 5000
N_P = 5000
E_TOT = 320000
D = 128

NT = 16
WIN = 80
EPT = E_TOT // NT
NWIN = EPT // WIN

AGG_ROWS = 5120
STRIPE = AGG_ROWS // NT
DEG_ROWS = 5120
DSTRIPE = DEG_ROWS // NT

_f32 = jnp.float32
_i32 = jnp.int32

_mesh = plsc.VectorSubcoreMesh(core_axis_name="c", subcore_axis_name="s")


@functools.partial(
    pl.kernel,
    out_type=jax.ShapeDtypeStruct((2 * DEG_ROWS,), _f32),
    mesh=_mesh,
    scratch_types=[
        pltpu.VMEM((NWIN, WIN), _i32),
        pltpu.VMEM((WIN,), _f32),
        pltpu.VMEM((DSTRIPE,), _f32),
        pltpu.VMEM_SHARED((DEG_ROWS,), _f32),
    ],
)
def _sc_degrees(eb_s, degs, idx_v, ones_v, z_v, deg_sh):
    cid = lax.axis_index("c")
    sid = lax.axis_index("s")

    for k in range(WIN // 16):
        ones_v[pl.ds(k * 16, 16)] = jnp.ones((16,), _f32)
    for k in range(DSTRIPE // 16):
        z_v[pl.ds(k * 16, 16)] = jnp.zeros((16,), _f32)

    base = sid * DSTRIPE
    pltpu.sync_copy(z_v, deg_sh.at[pl.ds(base, DSTRIPE)])
    pltpu.sync_copy(eb_s.at[cid, sid], idx_v)

    plsc.subcore_barrier()

    def body(w, carry):
        pltpu.sync_copy(ones_v, deg_sh.at[idx_v.at[w]], add=True)
        return carry

    lax.fori_loop(0, NWIN, body, 0)

    plsc.subcore_barrier()

    pltpu.sync_copy(deg_sh.at[pl.ds(base, DSTRIPE)], z_v)
    pltpu.sync_copy(z_v, degs.at[pl.ds(cid * DEG_ROWS + base, DSTRIPE)])


@functools.partial(
    pl.kernel,
    out_type=jax.ShapeDtypeStruct((2, AGG_ROWS, D), _f32),
    mesh=_mesh,
    scratch_types=[
        pltpu.VMEM((NWIN, WIN), _i32),
        pltpu.VMEM((NWIN, WIN), _i32),
        pltpu.VMEM((WIN, D), _f32),
        pltpu.VMEM((WIN, D), _f32),
        pltpu.VMEM((16, D), _f32),
        pltpu.VMEM_SHARED((AGG_ROWS, D), _f32),
        pltpu.SemaphoreType.DMA,
        pltpu.SemaphoreType.DMA,
    ],
)
def _sc_layer(y_cat, eb_g, eb_s, out,
              gidx, sidx, rows0, rows1, zblk, agg, sem0, sem1):
    cid = lax.axis_index("c")
    sid = lax.axis_index("s")

    for r in range(16):
        for c in range(D // 16):
            zblk[r, pl.ds(c * 16, 16)] = jnp.zeros((16,), _f32)

    base = sid * STRIPE
    for k in range(STRIPE // 16):
        pltpu.sync_copy(zblk, agg.at[pl.ds(base + k * 16, 16)])
    rem = STRIPE % 16
    if rem:
        pltpu.sync_copy(zblk.at[pl.ds(0, rem)],
                        agg.at[pl.ds(base + STRIPE - rem, rem)])

    pltpu.sync_copy(eb_g.at[cid, sid], gidx)
    pltpu.sync_copy(eb_s.at[cid, sid], sidx)

    plsc.subcore_barrier()

    def body(w, carry):
        pltpu.sync_copy(y_cat.at[gidx.at[w]], rows0)
        pltpu.sync_copy(rows0, agg.at[sidx.at[w]], add=True)
        return carry

    lax.fori_loop(0, NWIN, body, 0)

    plsc.subcore_barrier()
    pltpu.sync_copy(agg.at[pl.ds(base, STRIPE)],
                    out.at[cid, pl.ds(base, STRIPE)])


def _tc_pre_body(xa_ref, da_ref, w1_ref, xp_ref, dp_ref, w2_ref, y_ref):
    rs_a = lax.rsqrt(jnp.maximum(da_ref[...], 1.0))
    rs_p = lax.rsqrt(jnp.maximum(dp_ref[...], 1.0))
    y_ref[0] = jnp.dot(xa_ref[...] * rs_a, w1_ref[...],
                       preferred_element_type=_f32)
    y_ref[1] = jnp.dot(xp_ref[...] * rs_p, w2_ref[...],
                       preferred_element_type=_f32)


_tc_pre = pl.pallas_call(
    _tc_pre_body,
    out_shape=jax.ShapeDtypeStruct((2, N_A, D), _f32),
)


def _tc_mid_body(agga_ref, da_ref, ba_ref, wa_ref,
                 aggp_ref, dp_ref, bp_ref, wp_ref, y_ref):
    rs_a = lax.rsqrt(jnp.maximum(da_ref[...], 1.0))
    rs_p = lax.rsqrt(jnp.maximum(dp_ref[...], 1.0))
    h_a = jax.nn.relu(agga_ref[...] * rs_a + ba_ref[...])
    h_p = jax.nn.relu(aggp_ref[...] * rs_p + bp_ref[...])
    y_ref[0] = jnp.dot(h_a * rs_a, wa_ref[...],
                       preferred_element_type=_f32)
    y_ref[1] = jnp.dot(h_p * rs_p, wp_ref[...],
                       preferred_element_type=_f32)


_tc_mid = pl.pallas_call(
    _tc_mid_body,
    out_shape=jax.ShapeDtypeStruct((2, N_A, D), _f32),
)


def _tc_out_body(agga_ref, da_ref, ba_ref, aggp_ref, dp_ref, bp_ref,
                 oa_ref, op_ref):
    rs_a = lax.rsqrt(jnp.maximum(da_ref[...], 1.0))
    rs_p = lax.rsqrt(jnp.maximum(dp_ref[...], 1.0))
    oa_ref[...] = agga_ref[...] * rs_a + ba_ref[...]
    op_ref[...] = aggp_ref[...] * rs_p + bp_ref[...]


_tc_out = pl.pallas_call(
    _tc_out_body,
    out_shape=(
        jax.ShapeDtypeStruct((N_A, D), _f32),
        jax.ShapeDtypeStruct((N_P, D), _f32),
    ),
)


def kernel(x_author, x_paper, edge_src, edge_dst,
           W1w, b1w, W1b, b1b, W2w, b2w, W2b, b2b, W3w, b3w, W3b, b3b):
    src = edge_src.astype(_i32).reshape(NT, NWIN, WIN)
    dst = edge_dst.astype(_i32).reshape(NT, NWIN, WIN)
    eb_g = jnp.stack([src, dst + N_A])
    eb_s = jnp.stack([dst, src])

    degs = _sc_degrees(eb_s)
    deg_p = degs[:N_P].reshape(N_P, 1)
    deg_a = degs[DEG_ROWS:DEG_ROWS + N_A].reshape(N_A, 1)

    b1w_ = b1w.reshape(1, D)
    b1b_ = b1b.reshape(1, D)
    b2w_ = b2w.reshape(1, D)
    b2b_ = b2b.reshape(1, D)
    b3w_ = b3w.reshape(1, D)
    b3b_ = b3b.reshape(1, D)

    y1 = _tc_pre(x_author, deg_a, W1w, x_paper, deg_p, W1b)
    agg1 = _sc_layer(y1.reshape(2 * N_A, D), eb_g, eb_s)
    agg_p1, agg_a1 = agg1[0], agg1[1]

    y2 = _tc_mid(agg_a1[:N_A], deg_a, b1b_, W2w,
                 agg_p1[:N_P], deg_p, b1w_, W2b)
    agg2 = _sc_layer(y2.reshape(2 * N_A, D), eb_g, eb_s)
    agg_p2, agg_a2 = agg2[0], agg2[1]

    y3 = _tc_mid(agg_a2[:N_A], deg_a, b2b_, W3w,
                 agg_p2[:N_P], deg_p, b2w_, W3b)
    agg3 = _sc_layer(y3.reshape(2 * N_A, D), eb_g, eb_s)
    agg_p3, agg_a3 = agg3[0], agg3[1]

    out_a, out_p = _tc_out(agg_a3[:N_A], deg_a, b3b_,
                           agg_p3[:N_P], deg_p, b3w_)
    return (out_a, out_p)


# speedup vs baseline: 7.8645x; 1.7074x over previous
import functools

import jax
import jax.numpy as jnp
from jax import lax
from jax.experimental import pallas as pl
from jax.experimental.pallas import tpu as pltpu
from jax.experimental.pallas import tpu_sc as plsc

N_A = 5000
N_P = 5000
E_TOT = 320000
D = 128

NT = 16
WIN = 80
EPT = E_TOT // NT
NWIN = EPT // WIN

AGG_ROWS = 5120
STRIPE = AGG_ROWS // NT
DEG_ROWS = 5120
DSTRIPE = DEG_ROWS // NT

_f32 = jnp.float32
_i32 = jnp.int32

_mesh = plsc.VectorSubcoreMesh(core_axis_name="c", subcore_axis_name="s")


@functools.partial(
    pl.kernel,
    out_type=jax.ShapeDtypeStruct((2 * DEG_ROWS,), _f32),
    mesh=_mesh,
    scratch_types=[
        pltpu.VMEM((NWIN, WIN), _i32),
        pltpu.VMEM((WIN,), _f32),
        pltpu.VMEM((DSTRIPE,), _f32),
        pltpu.VMEM_SHARED((DEG_ROWS,), _f32),
    ],
)
def _sc_degrees(eb_s, degs, idx_v, ones_v, z_v, deg_sh):
    cid = lax.axis_index("c")
    sid = lax.axis_index("s")

    for k in range(WIN // 16):
        ones_v[pl.ds(k * 16, 16)] = jnp.ones((16,), _f32)
    for k in range(DSTRIPE // 16):
        z_v[pl.ds(k * 16, 16)] = jnp.zeros((16,), _f32)

    base = sid * DSTRIPE
    pltpu.sync_copy(z_v, deg_sh.at[pl.ds(base, DSTRIPE)])
    pltpu.sync_copy(eb_s.at[cid, sid], idx_v)

    plsc.subcore_barrier()

    def body(w, carry):
        pltpu.sync_copy(ones_v, deg_sh.at[idx_v.at[w]], add=True)
        return carry

    lax.fori_loop(0, NWIN, body, 0)

    plsc.subcore_barrier()

    pltpu.sync_copy(deg_sh.at[pl.ds(base, DSTRIPE)], z_v)
    pltpu.sync_copy(z_v, degs.at[pl.ds(cid * DEG_ROWS + base, DSTRIPE)])


@functools.partial(
    pl.kernel,
    out_type=jax.ShapeDtypeStruct((2, AGG_ROWS, D), _f32),
    mesh=_mesh,
    scratch_types=[
        pltpu.VMEM((NWIN, WIN), _i32),
        pltpu.VMEM((NWIN, WIN), _i32),
        pltpu.VMEM((WIN, D), _f32),
        pltpu.VMEM((WIN, D), _f32),
        pltpu.VMEM((16, D), _f32),
        pltpu.VMEM_SHARED((AGG_ROWS, D), _f32),
        pltpu.SemaphoreType.DMA,
        pltpu.SemaphoreType.DMA,
    ],
)
def _sc_layer(y_cat, eb_g, eb_s, out,
              gidx, sidx, rows0, rows1, zblk, agg, sem0, sem1):
    cid = lax.axis_index("c")
    sid = lax.axis_index("s")

    for r in range(16):
        for c in range(D // 16):
            zblk[r, pl.ds(c * 16, 16)] = jnp.zeros((16,), _f32)

    base = sid * STRIPE
    for k in range(STRIPE // 16):
        pltpu.sync_copy(zblk, agg.at[pl.ds(base + k * 16, 16)])
    rem = STRIPE % 16
    if rem:
        pltpu.sync_copy(zblk.at[pl.ds(0, rem)],
                        agg.at[pl.ds(base + STRIPE - rem, rem)])

    pltpu.sync_copy(eb_g.at[cid, sid], gidx)
    pltpu.sync_copy(eb_s.at[cid, sid], sidx)

    plsc.subcore_barrier()

    bufs = ((rows0, sem0), (rows1, sem1))
    pltpu.async_copy(y_cat.at[gidx.at[0]], rows0, sem0)
    pltpu.async_copy(y_cat.at[gidx.at[1]], rows1, sem1)

    def body(g, carry):
        for b, (rbuf, sem) in enumerate(bufs):
            w = 2 * g + b
            pltpu.make_async_copy(y_cat.at[gidx.at[w]], rbuf, sem).wait()
            pltpu.sync_copy(rbuf, agg.at[sidx.at[w]], add=True)
            nxt = w + 2

            @pl.when(nxt < NWIN)
            def _():
                pltpu.async_copy(y_cat.at[gidx.at[nxt]], rbuf, sem)

        return carry

    lax.fori_loop(0, NWIN // 2, body, 0)

    plsc.subcore_barrier()
    pltpu.sync_copy(agg.at[pl.ds(base, STRIPE)],
                    out.at[cid, pl.ds(base, STRIPE)])


def _tc_pre_body(xa_ref, da_ref, w1_ref, xp_ref, dp_ref, w2_ref, y_ref):
    rs_a = lax.rsqrt(jnp.maximum(da_ref[...], 1.0))
    rs_p = lax.rsqrt(jnp.maximum(dp_ref[...], 1.0))
    y_ref[0] = jnp.dot(xa_ref[...] * rs_a, w1_ref[...],
                       preferred_element_type=_f32)
    y_ref[1] = jnp.dot(xp_ref[...] * rs_p, w2_ref[...],
                       preferred_element_type=_f32)


_tc_pre = pl.pallas_call(
    _tc_pre_body,
    out_shape=jax.ShapeDtypeStruct((2, N_A, D), _f32),
)


def _tc_mid_body(agga_ref, da_ref, ba_ref, wa_ref,
                 aggp_ref, dp_ref, bp_ref, wp_ref, y_ref):
    rs_a = lax.rsqrt(jnp.maximum(da_ref[...], 1.0))
    rs_p = lax.rsqrt(jnp.maximum(dp_ref[...], 1.0))
    h_a = jax.nn.relu(agga_ref[...] * rs_a + ba_ref[...])
    h_p = jax.nn.relu(aggp_ref[...] * rs_p + bp_ref[...])
    y_ref[0] = jnp.dot(h_a * rs_a, wa_ref[...],
                       preferred_element_type=_f32)
    y_ref[1] = jnp.dot(h_p * rs_p, wp_ref[...],
                       preferred_element_type=_f32)


_tc_mid = pl.pallas_call(
    _tc_mid_body,
    out_shape=jax.ShapeDtypeStruct((2, N_A, D), _f32),
)


def _tc_out_body(agga_ref, da_ref, ba_ref, aggp_ref, dp_ref, bp_ref,
                 oa_ref, op_ref):
    rs_a = lax.rsqrt(jnp.maximum(da_ref[...], 1.0))
    rs_p = lax.rsqrt(jnp.maximum(dp_ref[...], 1.0))
    oa_ref[...] = agga_ref[...] * rs_a + ba_ref[...]
    op_ref[...] = aggp_ref[...] * rs_p + bp_ref[...]


_tc_out = pl.pallas_call(
    _tc_out_body,
    out_shape=(
        jax.ShapeDtypeStruct((N_A, D), _f32),
        jax.ShapeDtypeStruct((N_P, D), _f32),
    ),
)


def kernel(x_author, x_paper, edge_src, edge_dst,
           W1w, b1w, W1b, b1b, W2w, b2w, W2b, b2b, W3w, b3w, W3b, b3b):
    src = edge_src.astype(_i32).reshape(NT, NWIN, WIN)
    dst = edge_dst.astype(_i32).reshape(NT, NWIN, WIN)
    eb_g = jnp.stack([src, dst + N_A])
    eb_s = jnp.stack([dst, src])

    degs = _sc_degrees(eb_s)
    deg_p = degs[:N_P].reshape(N_P, 1)
    deg_a = degs[DEG_ROWS:DEG_ROWS + N_A].reshape(N_A, 1)

    b1w_ = b1w.reshape(1, D)
    b1b_ = b1b.reshape(1, D)
    b2w_ = b2w.reshape(1, D)
    b2b_ = b2b.reshape(1, D)
    b3w_ = b3w.reshape(1, D)
    b3b_ = b3b.reshape(1, D)

    y1 = _tc_pre(x_author, deg_a, W1w, x_paper, deg_p, W1b)
    agg1 = _sc_layer(y1.reshape(2 * N_A, D), eb_g, eb_s)
    agg_p1, agg_a1 = agg1[0], agg1[1]

    y2 = _tc_mid(agg_a1[:N_A], deg_a, b1b_, W2w,
                 agg_p1[:N_P], deg_p, b1w_, W2b)
    agg2 = _sc_layer(y2.reshape(2 * N_A, D), eb_g, eb_s)
    agg_p2, agg_a2 = agg2[0], agg2[1]

    y3 = _tc_mid(agg_a2[:N_A], deg_a, b2b_, W3w,
                 agg_p2[:N_P], deg_p, b2w_, W3b)
    agg3 = _sc_layer(y3.reshape(2 * N_A, D), eb_g, eb_s)
    agg_p3, agg_a3 = agg3[0], agg3[1]

    out_a, out_p = _tc_out(agg_a3[:N_A], deg_a, b3b_,
                           agg_p3[:N_P], deg_p, b3w_)
    return (out_a, out_p)


# speedup vs baseline: 8.4983x; 1.0806x over previous
import functools

import jax
import jax.numpy as jnp
from jax import lax
from jax.experimental import pallas as pl
from jax.experimental.pallas import tpu as pltpu
from jax.experimental.pallas import tpu_sc as plsc

N_A = 5000
N_P = 5000
E_TOT = 320000
D = 128

NT = 16
WIN = 80
EPT = E_TOT // NT
NWIN = EPT // WIN

AGG_ROWS = 5120
STRIPE = AGG_ROWS // NT
DEG_ROWS = 5120
DSTRIPE = DEG_ROWS // NT

_f32 = jnp.float32
_i32 = jnp.int32

_mesh = plsc.VectorSubcoreMesh(core_axis_name="c", subcore_axis_name="s")


@functools.partial(
    pl.kernel,
    out_type=jax.ShapeDtypeStruct((2 * DEG_ROWS,), _f32),
    mesh=_mesh,
    scratch_types=[
        pltpu.VMEM((NWIN, WIN), _i32),
        pltpu.VMEM((1, WIN), _i32),
        pltpu.VMEM((WIN,), _f32),
        pltpu.VMEM((DSTRIPE,), _f32),
        pltpu.VMEM_SHARED((DEG_ROWS,), _f32),
    ],
)
def _sc_degrees(eb_comb, degs, comb_v, sidx_v, ones_v, z_v, deg_sh):
    cid = lax.axis_index("c")
    sid = lax.axis_index("s")

    for k in range(WIN // 16):
        ones_v[pl.ds(k * 16, 16)] = jnp.ones((16,), _f32)
    for k in range(DSTRIPE // 16):
        z_v[pl.ds(k * 16, 16)] = jnp.zeros((16,), _f32)

    base = sid * DSTRIPE
    pltpu.sync_copy(z_v, deg_sh.at[pl.ds(base, DSTRIPE)])
    pltpu.sync_copy(eb_comb.at[cid, sid], comb_v)

    plsc.subcore_barrier()

    def body(w, carry):
        for k in range(WIN // 16):
            v = comb_v[w, pl.ds(16 * k, 16)]
            sidx_v[0, pl.ds(16 * k, 16)] = lax.shift_right_logical(v, 16)
        pltpu.sync_copy(ones_v, deg_sh.at[sidx_v.at[0]], add=True)
        return carry

    lax.fori_loop(0, NWIN, body, 0)

    plsc.subcore_barrier()

    pltpu.sync_copy(deg_sh.at[pl.ds(base, DSTRIPE)], z_v)
    pltpu.sync_copy(z_v, degs.at[pl.ds(cid * DEG_ROWS + base, DSTRIPE)])


@functools.partial(
    pl.kernel,
    out_type=jax.ShapeDtypeStruct((2, AGG_ROWS, D), _f32),
    mesh=_mesh,
    scratch_types=[
        pltpu.VMEM((NWIN, WIN), _i32),
        pltpu.VMEM((5, WIN), _i32),
        pltpu.VMEM((5, WIN), _i32),
        pltpu.VMEM((5, WIN, D), _f32),
        pltpu.VMEM((16, D), _f32),
        pltpu.VMEM_SHARED((AGG_ROWS, D), _f32),
        pltpu.SemaphoreType.DMA,
        pltpu.SemaphoreType.DMA,
        pltpu.SemaphoreType.DMA,
        pltpu.SemaphoreType.DMA,
        pltpu.SemaphoreType.DMA,
        pltpu.SemaphoreType.DMA,
        pltpu.SemaphoreType.DMA,
        pltpu.SemaphoreType.DMA,
        pltpu.SemaphoreType.DMA,
        pltpu.SemaphoreType.DMA,
    ],
)
def _sc_layer(y_cat, eb_comb, out,
              comb_v, gwork, swork, rows, zblk, agg,
              gs0, gs1, gs2, gs3, gs4, ss0, ss1, ss2, ss3, ss4):
    cid = lax.axis_index("c")
    sid = lax.axis_index("s")

    for r in range(16):
        for c in range(D // 16):
            zblk[r, pl.ds(c * 16, 16)] = jnp.zeros((16,), _f32)

    base = sid * STRIPE
    for k in range(STRIPE // 16):
        pltpu.sync_copy(zblk, agg.at[pl.ds(base + k * 16, 16)])
    rem = STRIPE % 16
    if rem:
        pltpu.sync_copy(zblk.at[pl.ds(0, rem)],
                        agg.at[pl.ds(base + STRIPE - rem, rem)])

    pltpu.sync_copy(eb_comb.at[cid, sid], comb_v)

    plsc.subcore_barrier()

    def unpack(w, b):
        for k in range(WIN // 16):
            v = comb_v[w, pl.ds(16 * k, 16)]
            gwork[b, pl.ds(16 * k, 16)] = lax.bitwise_and(
                v, jnp.int32(0xFFFF))
            swork[b, pl.ds(16 * k, 16)] = lax.shift_right_logical(v, 16)

    NB = 5
    gsems = (gs0, gs1, gs2, gs3, gs4)
    ssems = (ss0, ss1, ss2, ss3, ss4)
    for b in range(NB):
        unpack(b, b)
        pltpu.async_copy(y_cat.at[gwork.at[b]], rows.at[b], gsems[b])

    def body(g, carry):
        w0 = NB * g
        for b in range(NB):
            pltpu.make_async_copy(y_cat.at[gwork.at[b]], rows.at[b],
                                  gsems[b]).wait()
            pltpu.async_copy(rows.at[b], agg.at[swork.at[b]], ssems[b],
                             add=True)
        for b in range(NB):
            w = w0 + b
            pltpu.make_async_copy(rows.at[b], agg.at[swork.at[b]],
                                  ssems[b]).wait()
            nxt = w + NB

            @pl.when(nxt < NWIN)
            def _():
                unpack(nxt, b)
                pltpu.async_copy(y_cat.at[gwork.at[b]], rows.at[b], gsems[b])

        return carry

    lax.fori_loop(0, NWIN // NB, body, 0)

    plsc.subcore_barrier()
    pltpu.sync_copy(agg.at[pl.ds(base, STRIPE)],
                    out.at[cid, pl.ds(base, STRIPE)])


def _tc_pre_body(xa_ref, da_ref, w1_ref, xp_ref, dp_ref, w2_ref, y_ref):
    rs_a = lax.rsqrt(jnp.maximum(da_ref[...], 1.0))
    rs_p = lax.rsqrt(jnp.maximum(dp_ref[...], 1.0))
    y_ref[0] = jnp.dot(xa_ref[...] * rs_a, w1_ref[...],
                       preferred_element_type=_f32)
    y_ref[1] = jnp.dot(xp_ref[...] * rs_p, w2_ref[...],
                       preferred_element_type=_f32)


_tc_pre = pl.pallas_call(
    _tc_pre_body,
    out_shape=jax.ShapeDtypeStruct((2, N_A, D), _f32),
)


def _tc_mid_body(agga_ref, da_ref, ba_ref, wa_ref,
                 aggp_ref, dp_ref, bp_ref, wp_ref, y_ref):
    rs_a = lax.rsqrt(jnp.maximum(da_ref[...], 1.0))
    rs_p = lax.rsqrt(jnp.maximum(dp_ref[...], 1.0))
    h_a = jax.nn.relu(agga_ref[...] * rs_a + ba_ref[...])
    h_p = jax.nn.relu(aggp_ref[...] * rs_p + bp_ref[...])
    y_ref[0] = jnp.dot(h_a * rs_a, wa_ref[...],
                       preferred_element_type=_f32)
    y_ref[1] = jnp.dot(h_p * rs_p, wp_ref[...],
                       preferred_element_type=_f32)


_tc_mid = pl.pallas_call(
    _tc_mid_body,
    out_shape=jax.ShapeDtypeStruct((2, N_A, D), _f32),
)


def _tc_out_body(agga_ref, da_ref, ba_ref, aggp_ref, dp_ref, bp_ref,
                 oa_ref, op_ref):
    rs_a = lax.rsqrt(jnp.maximum(da_ref[...], 1.0))
    rs_p = lax.rsqrt(jnp.maximum(dp_ref[...], 1.0))
    oa_ref[...] = agga_ref[...] * rs_a + ba_ref[...]
    op_ref[...] = aggp_ref[...] * rs_p + bp_ref[...]


_tc_out = pl.pallas_call(
    _tc_out_body,
    out_shape=(
        jax.ShapeDtypeStruct((N_A, D), _f32),
        jax.ShapeDtypeStruct((N_P, D), _f32),
    ),
)


def kernel(x_author, x_paper, edge_src, edge_dst,
           W1w, b1w, W1b, b1b, W2w, b2w, W2b, b2b, W3w, b3w, W3b, b3b):
    src = edge_src.astype(_i32).reshape(NT, NWIN, WIN)
    dst = edge_dst.astype(_i32).reshape(NT, NWIN, WIN)
    eb_comb = jnp.stack([src + (dst << 16), (dst + N_A) + (src << 16)])

    degs = _sc_degrees(eb_comb)
    deg_p = degs[:N_P].reshape(N_P, 1)
    deg_a = degs[DEG_ROWS:DEG_ROWS + N_A].reshape(N_A, 1)

    b1w_ = b1w.reshape(1, D)
    b1b_ = b1b.reshape(1, D)
    b2w_ = b2w.reshape(1, D)
    b2b_ = b2b.reshape(1, D)
    b3w_ = b3w.reshape(1, D)
    b3b_ = b3b.reshape(1, D)

    y1 = _tc_pre(x_author, deg_a, W1w, x_paper, deg_p, W1b)
    agg1 = _sc_layer(y1.reshape(2 * N_A, D), eb_comb)
    agg_p1, agg_a1 = agg1[0], agg1[1]

    y2 = _tc_mid(agg_a1[:N_A], deg_a, b1b_, W2w,
                 agg_p1[:N_P], deg_p, b1w_, W2b)
    agg2 = _sc_layer(y2.reshape(2 * N_A, D), eb_comb)
    agg_p2, agg_a2 = agg2[0], agg2[1]

    y3 = _tc_mid(agg_a2[:N_A], deg_a, b2b_, W3w,
                 agg_p2[:N_P], deg_p, b2w_, W3b)
    agg3 = _sc_layer(y3.reshape(2 * N_A, D), eb_comb)
    agg_p3, agg_a3 = agg3[0], agg3[1]

    out_a, out_p = _tc_out(agg_a3[:N_A], deg_a, b3b_,
                           agg_p3[:N_P], deg_p, b3w_)
    return (out_a, out_p)


# speedup vs baseline: 8.6370x; 1.0163x over previous
import functools

import jax
import jax.numpy as jnp
from jax import lax
from jax.experimental import pallas as pl
from jax.experimental.pallas import tpu as pltpu
from jax.experimental.pallas import tpu_sc as plsc

N_A = 5000
N_P = 5000
E_TOT = 320000
D = 128

NT = 16
WIN = 100
CHUNKS = list(range(0, WIN - 15, 16)) + ([WIN - 16] if WIN % 16 else [])
EPT = E_TOT // NT
NWIN = EPT // WIN

AGG_ROWS = 5120
STRIPE = AGG_ROWS // NT
DEG_ROWS = 5120
DSTRIPE = DEG_ROWS // NT

_f32 = jnp.float32
_i32 = jnp.int32

_mesh = plsc.VectorSubcoreMesh(core_axis_name="c", subcore_axis_name="s")


@functools.partial(
    pl.kernel,
    out_type=jax.ShapeDtypeStruct((2 * DEG_ROWS,), _f32),
    mesh=_mesh,
    scratch_types=[
        pltpu.VMEM((NWIN, WIN), _i32),
        pltpu.VMEM((4, WIN), _i32),
        pltpu.VMEM((WIN,), _f32),
        pltpu.VMEM((DSTRIPE,), _f32),
        pltpu.VMEM_SHARED((DEG_ROWS,), _f32),
        pltpu.SemaphoreType.DMA,
        pltpu.SemaphoreType.DMA,
        pltpu.SemaphoreType.DMA,
        pltpu.SemaphoreType.DMA,
    ],
)
def _sc_degrees(eb_comb, degs, comb_v, sidx_v, ones_v, z_v, deg_sh,
                ds0, ds1, ds2, ds3):
    cid = lax.axis_index("c")
    sid = lax.axis_index("s")

    for c in CHUNKS:
        ones_v[pl.ds(c, 16)] = jnp.ones((16,), _f32)
    for k in range(DSTRIPE // 16):
        z_v[pl.ds(k * 16, 16)] = jnp.zeros((16,), _f32)

    base = sid * DSTRIPE
    pltpu.sync_copy(z_v, deg_sh.at[pl.ds(base, DSTRIPE)])
    pltpu.sync_copy(eb_comb.at[cid, sid], comb_v)

    plsc.subcore_barrier()

    dsems = (ds0, ds1, ds2, ds3)

    def body(g, carry):
        for b in range(4):
            w = 4 * g + b

            @pl.when(g > 0)
            def _():
                pltpu.make_async_copy(ones_v, deg_sh.at[sidx_v.at[b]],
                                      dsems[b]).wait()
            for c in CHUNKS:
                v = comb_v[w, pl.ds(c, 16)]
                sidx_v[b, pl.ds(c, 16)] = lax.shift_right_logical(v, 16)
            pltpu.async_copy(ones_v, deg_sh.at[sidx_v.at[b]], dsems[b],
                             add=True)
        return carry

    lax.fori_loop(0, NWIN // 4, body, 0)
    for b in range(4):
        pltpu.make_async_copy(ones_v, deg_sh.at[sidx_v.at[b]], dsems[b]).wait()

    plsc.subcore_barrier()

    pltpu.sync_copy(deg_sh.at[pl.ds(base, DSTRIPE)], z_v)
    pltpu.sync_copy(z_v, degs.at[pl.ds(cid * DEG_ROWS + base, DSTRIPE)])


@functools.partial(
    pl.kernel,
    out_type=jax.ShapeDtypeStruct((2, AGG_ROWS, D), _f32),
    mesh=_mesh,
    scratch_types=[
        pltpu.VMEM((NWIN, WIN), _i32),
        pltpu.VMEM((4, WIN), _i32),
        pltpu.VMEM((4, WIN), _i32),
        pltpu.VMEM((4, WIN, D), _f32),
        pltpu.VMEM((16, D), _f32),
        pltpu.VMEM_SHARED((AGG_ROWS, D), _f32),
        pltpu.SemaphoreType.DMA,
        pltpu.SemaphoreType.DMA,
        pltpu.SemaphoreType.DMA,
        pltpu.SemaphoreType.DMA,
        pltpu.SemaphoreType.DMA,
        pltpu.SemaphoreType.DMA,
        pltpu.SemaphoreType.DMA,
        pltpu.SemaphoreType.DMA,
    ],
)
def _sc_layer(y_cat, eb_comb, out,
              comb_v, gwork, swork, rows, zblk, agg,
              gs0, gs1, gs2, gs3, ss0, ss1, ss2, ss3):
    cid = lax.axis_index("c")
    sid = lax.axis_index("s")

    for r in range(16):
        for c in range(D // 16):
            zblk[r, pl.ds(c * 16, 16)] = jnp.zeros((16,), _f32)

    base = sid * STRIPE
    for k in range(STRIPE // 16):
        pltpu.sync_copy(zblk, agg.at[pl.ds(base + k * 16, 16)])
    rem = STRIPE % 16
    if rem:
        pltpu.sync_copy(zblk.at[pl.ds(0, rem)],
                        agg.at[pl.ds(base + STRIPE - rem, rem)])

    pltpu.sync_copy(eb_comb.at[cid, sid], comb_v)

    plsc.subcore_barrier()

    def unpack(w, b):
        for c in CHUNKS:
            v = comb_v[w, pl.ds(c, 16)]
            gwork[b, pl.ds(c, 16)] = lax.bitwise_and(v, jnp.int32(0xFFFF))
            swork[b, pl.ds(c, 16)] = lax.shift_right_logical(v, 16)

    NB = 4
    gsems = (gs0, gs1, gs2, gs3)
    ssems = (ss0, ss1, ss2, ss3)
    for b in range(NB):
        unpack(b, b)
        pltpu.async_copy(y_cat.at[gwork.at[b]], rows.at[b], gsems[b])

    def body(g, carry):
        w0 = NB * g
        for b in range(NB):
            pltpu.make_async_copy(y_cat.at[gwork.at[b]], rows.at[b],
                                  gsems[b]).wait()
            pltpu.async_copy(rows.at[b], agg.at[swork.at[b]], ssems[b],
                             add=True)
        for b in range(NB):
            w = w0 + b
            pltpu.make_async_copy(rows.at[b], agg.at[swork.at[b]],
                                  ssems[b]).wait()
            nxt = w + NB

            @pl.when(nxt < NWIN)
            def _():
                unpack(nxt, b)
                pltpu.async_copy(y_cat.at[gwork.at[b]], rows.at[b], gsems[b])

        return carry

    lax.fori_loop(0, NWIN // NB, body, 0)

    plsc.subcore_barrier()
    pltpu.sync_copy(agg.at[pl.ds(base, STRIPE)],
                    out.at[cid, pl.ds(base, STRIPE)])


def _tc_pre_body(xa_ref, da_ref, w1_ref, xp_ref, dp_ref, w2_ref, y_ref):
    rs_a = lax.rsqrt(jnp.maximum(da_ref[...], 1.0))
    rs_p = lax.rsqrt(jnp.maximum(dp_ref[...], 1.0))
    y_ref[0] = jnp.dot(xa_ref[...] * rs_a, w1_ref[...],
                       preferred_element_type=_f32)
    y_ref[1] = jnp.dot(xp_ref[...] * rs_p, w2_ref[...],
                       preferred_element_type=_f32)


_tc_pre = pl.pallas_call(
    _tc_pre_body,
    out_shape=jax.ShapeDtypeStruct((2, N_A, D), _f32),
)


def _tc_mid_body(agga_ref, da_ref, ba_ref, wa_ref,
                 aggp_ref, dp_ref, bp_ref, wp_ref, y_ref):
    rs_a = lax.rsqrt(jnp.maximum(da_ref[...], 1.0))
    rs_p = lax.rsqrt(jnp.maximum(dp_ref[...], 1.0))
    h_a = jax.nn.relu(agga_ref[...] * rs_a + ba_ref[...])
    h_p = jax.nn.relu(aggp_ref[...] * rs_p + bp_ref[...])
    y_ref[0] = jnp.dot(h_a * rs_a, wa_ref[...],
                       preferred_element_type=_f32)
    y_ref[1] = jnp.dot(h_p * rs_p, wp_ref[...],
                       preferred_element_type=_f32)


_tc_mid = pl.pallas_call(
    _tc_mid_body,
    out_shape=jax.ShapeDtypeStruct((2, N_A, D), _f32),
)


def _tc_out_body(agga_ref, da_ref, ba_ref, aggp_ref, dp_ref, bp_ref,
                 oa_ref, op_ref):
    rs_a = lax.rsqrt(jnp.maximum(da_ref[...], 1.0))
    rs_p = lax.rsqrt(jnp.maximum(dp_ref[...], 1.0))
    oa_ref[...] = agga_ref[...] * rs_a + ba_ref[...]
    op_ref[...] = aggp_ref[...] * rs_p + bp_ref[...]


_tc_out = pl.pallas_call(
    _tc_out_body,
    out_shape=(
        jax.ShapeDtypeStruct((N_A, D), _f32),
        jax.ShapeDtypeStruct((N_P, D), _f32),
    ),
)


def kernel(x_author, x_paper, edge_src, edge_dst,
           W1w, b1w, W1b, b1b, W2w, b2w, W2b, b2b, W3w, b3w, W3b, b3b):
    src = edge_src.astype(_i32).reshape(NT, NWIN, WIN)
    dst = edge_dst.astype(_i32).reshape(NT, NWIN, WIN)
    eb_comb = jnp.stack([src + (dst << 16), (dst + N_A) + (src << 16)])

    degs = _sc_degrees(eb_comb)
    deg_p = degs[:N_P].reshape(N_P, 1)
    deg_a = degs[DEG_ROWS:DEG_ROWS + N_A].reshape(N_A, 1)

    b1w_ = b1w.reshape(1, D)
    b1b_ = b1b.reshape(1, D)
    b2w_ = b2w.reshape(1, D)
    b2b_ = b2b.reshape(1, D)
    b3w_ = b3w.reshape(1, D)
    b3b_ = b3b.reshape(1, D)

    y1 = _tc_pre(x_author, deg_a, W1w, x_paper, deg_p, W1b)
    agg1 = _sc_layer(y1.reshape(2 * N_A, D), eb_comb)
    agg_p1, agg_a1 = agg1[0], agg1[1]

    y2 = _tc_mid(agg_a1[:N_A], deg_a, b1b_, W2w,
                 agg_p1[:N_P], deg_p, b1w_, W2b)
    agg2 = _sc_layer(y2.reshape(2 * N_A, D), eb_comb)
    agg_p2, agg_a2 = agg2[0], agg2[1]

    y3 = _tc_mid(agg_a2[:N_A], deg_a, b2b_, W3w,
                 agg_p2[:N_P], deg_p, b2w_, W3b)
    agg3 = _sc_layer(y3.reshape(2 * N_A, D), eb_comb)
    agg_p3, agg_a3 = agg3[0], agg3[1]

    out_a, out_p = _tc_out(agg_a3[:N_A], deg_a, b3b_,
                           agg_p3[:N_P], deg_p, b3w_)
    return (out_a, out_p)


# speedup vs baseline: 10.8795x; 1.2596x over previous
import functools

import jax
import jax.numpy as jnp
from jax import lax
from jax.experimental import pallas as pl
from jax.experimental.pallas import tpu as pltpu
from jax.experimental.pallas import tpu_sc as plsc

N_A = 5000
N_P = 5000
E_TOT = 320000
D = 128

NPAD = 5120
NT = 16
WIN = 100
EPT = E_TOT // NT
NWIN = EPT // WIN
CHUNKS = list(range(0, WIN - 15, 16)) + ([WIN - 16] if WIN % 16 else [])

DEG_ROWS = NPAD
DSTRIPE = DEG_ROWS // NT

ACH = 256
NPASS = (NPAD // 2) // ACH
ACELLS = (ACH + 1) * NPAD

_f32 = jnp.float32
_i32 = jnp.int32

_mesh = plsc.VectorSubcoreMesh(core_axis_name="c", subcore_axis_name="s")


@functools.partial(
    pl.kernel,
    out_type=jax.ShapeDtypeStruct((2 * DEG_ROWS,), _f32),
    mesh=_mesh,
    scratch_types=[
        pltpu.VMEM((NWIN, WIN), _i32),
        pltpu.VMEM((4, WIN), _i32),
        pltpu.VMEM((WIN,), _f32),
        pltpu.VMEM((DSTRIPE,), _f32),
        pltpu.VMEM_SHARED((DEG_ROWS,), _f32),
        pltpu.SemaphoreType.DMA,
        pltpu.SemaphoreType.DMA,
        pltpu.SemaphoreType.DMA,
        pltpu.SemaphoreType.DMA,
    ],
)
def _sc_degrees(eb_ad, degs, comb_v, sidx_v, ones_v, z_v, deg_sh,
                ds0, ds1, ds2, ds3):
    cid = lax.axis_index("c")
    sid = lax.axis_index("s")

    for c in CHUNKS:
        ones_v[pl.ds(c, 16)] = jnp.ones((16,), _f32)
    for k in range(DSTRIPE // 16):
        z_v[pl.ds(k * 16, 16)] = jnp.zeros((16,), _f32)

    base = sid * DSTRIPE
    pltpu.sync_copy(z_v, deg_sh.at[pl.ds(base, DSTRIPE)])
    pltpu.sync_copy(eb_ad.at[sid], comb_v)

    plsc.subcore_barrier()

    dsems = (ds0, ds1, ds2, ds3)
    use_hi = cid == 0

    def body(g, carry):
        for b in range(4):
            w = 4 * g + b

            @pl.when(g > 0)
            def _():
                pltpu.make_async_copy(ones_v, deg_sh.at[sidx_v.at[b]],
                                      dsems[b]).wait()

            for c in CHUNKS:
                v = comb_v[w, pl.ds(c, 16)]
                hi = lax.shift_right_logical(v, 16)
                lo = lax.bitwise_and(v, jnp.int32(0xFFFF))
                sidx_v[b, pl.ds(c, 16)] = jnp.where(use_hi, hi, lo)
            pltpu.async_copy(ones_v, deg_sh.at[sidx_v.at[b]], dsems[b],
                             add=True)
        return carry

    lax.fori_loop(0, NWIN // 4, body, 0)
    for b in range(4):
        pltpu.make_async_copy(ones_v, deg_sh.at[sidx_v.at[b]], dsems[b]).wait()

    plsc.subcore_barrier()

    pltpu.sync_copy(deg_sh.at[pl.ds(base, DSTRIPE)], z_v)
    pltpu.sync_copy(z_v, degs.at[pl.ds(cid * DEG_ROWS + base, DSTRIPE)])


@functools.partial(
    pl.kernel,
    out_type=jax.ShapeDtypeStruct((NPAD * NPAD,), _f32),
    mesh=_mesh,
    scratch_types=[
        pltpu.VMEM((NWIN, WIN), _i32),
        pltpu.VMEM((4, WIN), _i32),
        pltpu.VMEM((WIN,), _f32),
        pltpu.VMEM((4 * NPAD,), _f32),
        pltpu.VMEM_SHARED((ACELLS,), _f32),
        pltpu.SemaphoreType.DMA,
        pltpu.SemaphoreType.DMA,
        pltpu.SemaphoreType.DMA,
        pltpu.SemaphoreType.DMA,
    ],
)
def _sc_build_a(eb_ad, a_out, comb_v, fidx, ones_v, zrow, chunk,
                cs0, cs1, cs2, cs3):
    cid = lax.axis_index("c")
    sid = lax.axis_index("s")

    for c in CHUNKS:
        ones_v[pl.ds(c, 16)] = jnp.ones((16,), _f32)

    def zbody(i, carry):
        zrow[pl.ds(16 * i, 16)] = jnp.zeros((16,), _f32)
        return carry

    lax.fori_loop(0, 4 * NPAD // 16, zbody, 0)

    pltpu.sync_copy(eb_ad.at[sid], comb_v)

    csems = (cs0, cs1, cs2, cs3)
    rows_per_tile = ACH // NT

    def pass_body(p, carry0):
        gbase = cid * (NPAD // 2) + p * ACH
        for k in range(4):
            pltpu.sync_copy(
                zrow, chunk.at[pl.ds((sid * rows_per_tile + 4 * k) * NPAD,
                                     4 * NPAD)])
        plsc.subcore_barrier()

        def wbody(g, carry):
            for b in range(4):
                w = 4 * g + b

                @pl.when(g > 0)
                def _():
                    pltpu.make_async_copy(ones_v, chunk.at[fidx.at[b]],
                                          csems[b]).wait()

                for c in CHUNKS:
                    v = comb_v[w, pl.ds(c, 16)]
                    srcv = lax.bitwise_and(v, jnp.int32(0xFFFF))
                    dstv = lax.shift_right_logical(v, 16)
                    rel = dstv - gbase
                    ok = jnp.logical_and(rel >= 0, rel < ACH)
                    rel = jnp.where(ok, rel, ACH)
                    fidx[b, pl.ds(c, 16)] = rel * NPAD + srcv
                pltpu.async_copy(ones_v, chunk.at[fidx.at[b]], csems[b],
                                 add=True)
            return carry

        lax.fori_loop(0, NWIN // 4, wbody, 0)
        for b in range(4):
            pltpu.make_async_copy(ones_v, chunk.at[fidx.at[b]],
                                  csems[b]).wait()
        plsc.subcore_barrier()

        pltpu.sync_copy(
            chunk.at[pl.ds(sid * rows_per_tile * NPAD, rows_per_tile * NPAD)],
            a_out.at[pl.ds((gbase + sid * rows_per_tile) * NPAD,
                           rows_per_tile * NPAD)])
        plsc.subcore_barrier()
        return carry0

    lax.fori_loop(0, NPASS, pass_body, 0)


def _tc_agg_body(a_ref, ya_ref, yp_ref, aggp_ref, agga_ref):
    i = pl.program_id(0)
    aggp_ref[...] = jnp.dot(a_ref[...], ya_ref[...],
                            preferred_element_type=_f32)
    contrib = lax.dot_general(a_ref[...], yp_ref[...],
                              (((0,), (0,)), ((), ())),
                              preferred_element_type=_f32)

    @pl.when(i == 0)
    def _():
        agga_ref[...] = contrib

    @pl.when(i > 0)
    def _():
        agga_ref[...] = agga_ref[...] + contrib


_tc_agg = pl.pallas_call(
    _tc_agg_body,
    grid=(NPAD // ACH,),
    in_specs=[
        pl.BlockSpec((ACH, NPAD), lambda i: (i, 0)),
        pl.BlockSpec((NPAD, D), lambda i: (0, 0)),
        pl.BlockSpec((ACH, D), lambda i: (i, 0)),
    ],
    out_specs=[
        pl.BlockSpec((ACH, D), lambda i: (i, 0)),
        pl.BlockSpec((NPAD, D), lambda i: (0, 0)),
    ],
    out_shape=(
        jax.ShapeDtypeStruct((NPAD, D), _f32),
        jax.ShapeDtypeStruct((NPAD, D), _f32),
    ),
)


def _tc_pre_body(xa_ref, da_ref, w1_ref, xp_ref, dp_ref, w2_ref, y_ref):
    rs_a = lax.rsqrt(jnp.maximum(da_ref[...], 1.0))
    rs_p = lax.rsqrt(jnp.maximum(dp_ref[...], 1.0))
    y_ref[0] = jnp.dot(xa_ref[...] * rs_a, w1_ref[...],
                       preferred_element_type=_f32)
    y_ref[1] = jnp.dot(xp_ref[...] * rs_p, w2_ref[...],
                       preferred_element_type=_f32)


_tc_pre = pl.pallas_call(
    _tc_pre_body,
    out_shape=jax.ShapeDtypeStruct((2, NPAD, D), _f32),
)


def _tc_mid_body(agga_ref, da_ref, ba_ref, wa_ref,
                 aggp_ref, dp_ref, bp_ref, wp_ref, y_ref):
    rs_a = lax.rsqrt(jnp.maximum(da_ref[...], 1.0))
    rs_p = lax.rsqrt(jnp.maximum(dp_ref[...], 1.0))
    h_a = jax.nn.relu(agga_ref[...] * rs_a + ba_ref[...])
    h_p = jax.nn.relu(aggp_ref[...] * rs_p + bp_ref[...])
    y_ref[0] = jnp.dot(h_a * rs_a, wa_ref[...],
                       preferred_element_type=_f32)
    y_ref[1] = jnp.dot(h_p * rs_p, wp_ref[...],
                       preferred_element_type=_f32)


_tc_mid = pl.pallas_call(
    _tc_mid_body,
    out_shape=jax.ShapeDtypeStruct((2, NPAD, D), _f32),
)


def _tc_out_body(agga_ref, da_ref, ba_ref, aggp_ref, dp_ref, bp_ref,
                 oa_ref, op_ref):
    rs_a = lax.rsqrt(jnp.maximum(da_ref[...], 1.0))
    rs_p = lax.rsqrt(jnp.maximum(dp_ref[...], 1.0))
    oa_ref[...] = agga_ref[...] * rs_a + ba_ref[...]
    op_ref[...] = aggp_ref[...] * rs_p + bp_ref[...]


_tc_out = pl.pallas_call(
    _tc_out_body,
    out_shape=(
        jax.ShapeDtypeStruct((NPAD, D), _f32),
        jax.ShapeDtypeStruct((NPAD, D), _f32),
    ),
)


def kernel(x_author, x_paper, edge_src, edge_dst,
           W1w, b1w, W1b, b1b, W2w, b2w, W2b, b2b, W3w, b3w, W3b, b3b):
    src = edge_src.astype(_i32).reshape(NT, NWIN, WIN)
    dst = edge_dst.astype(_i32).reshape(NT, NWIN, WIN)
    eb_ad = src + (dst << 16)

    degs = _sc_degrees(eb_ad)
    deg_p = degs[:DEG_ROWS].reshape(NPAD, 1)
    deg_a = degs[DEG_ROWS:].reshape(NPAD, 1)

    a_mat = _sc_build_a(eb_ad).reshape(NPAD, NPAD)

    xa = jnp.pad(x_author, ((0, NPAD - N_A), (0, 0)))
    xp = jnp.pad(x_paper, ((0, NPAD - N_P), (0, 0)))

    b1w_ = b1w.reshape(1, D)
    b1b_ = b1b.reshape(1, D)
    b2w_ = b2w.reshape(1, D)
    b2b_ = b2b.reshape(1, D)
    b3w_ = b3w.reshape(1, D)
    b3b_ = b3b.reshape(1, D)

    y1 = _tc_pre(xa, deg_a, W1w, xp, deg_p, W1b)
    agg_p1, agg_a1 = _tc_agg(a_mat, y1[0], y1[1])

    y2 = _tc_mid(agg_a1, deg_a, b1b_, W2w, agg_p1, deg_p, b1w_, W2b)
    agg_p2, agg_a2 = _tc_agg(a_mat, y2[0], y2[1])

    y3 = _tc_mid(agg_a2, deg_a, b2b_, W3w, agg_p2, deg_p, b2w_, W3b)
    agg_p3, agg_a3 = _tc_agg(a_mat, y3[0], y3[1])

    out_a, out_p = _tc_out(agg_a3, deg_a, b3b_, agg_p3, deg_p, b3w_)
    return (out_a[:N_A], out_p[:N_P])


# speedup vs baseline: 11.0141x; 1.0124x over previous
import functools

import jax
import jax.numpy as jnp
from jax import lax
from jax.experimental import pallas as pl
from jax.experimental.pallas import tpu as pltpu
from jax.experimental.pallas import tpu_sc as plsc

N_A = 5000
N_P = 5000
E_TOT = 320000
D = 128

NPAD = 5120
NT = 16
WIN = 100
EPT = E_TOT // NT
NWIN = EPT // WIN
CHUNKS = list(range(0, WIN - 15, 16)) + ([WIN - 16] if WIN % 16 else [])

DEG_ROWS = NPAD
DSTRIPE = DEG_ROWS // NT

ACH = 256
NPASS = (NPAD // 2) // ACH
ACELLS = (ACH + 1) * NPAD

_f32 = jnp.float32
_i32 = jnp.int32

_mesh = plsc.VectorSubcoreMesh(core_axis_name="c", subcore_axis_name="s")


@functools.partial(
    pl.kernel,
    out_type=jax.ShapeDtypeStruct((2 * DEG_ROWS,), _f32),
    mesh=_mesh,
    scratch_types=[
        pltpu.VMEM((NWIN, WIN), _i32),
        pltpu.VMEM((4, WIN), _i32),
        pltpu.VMEM((WIN,), _f32),
        pltpu.VMEM((DSTRIPE,), _f32),
        pltpu.VMEM_SHARED((DEG_ROWS,), _f32),
        pltpu.SemaphoreType.DMA,
        pltpu.SemaphoreType.DMA,
        pltpu.SemaphoreType.DMA,
        pltpu.SemaphoreType.DMA,
    ],
)
def _sc_degrees(eb_ad, degs, comb_v, sidx_v, ones_v, z_v, deg_sh,
                ds0, ds1, ds2, ds3):
    cid = lax.axis_index("c")
    sid = lax.axis_index("s")

    for c in CHUNKS:
        ones_v[pl.ds(c, 16)] = jnp.ones((16,), _f32)
    for k in range(DSTRIPE // 16):
        z_v[pl.ds(k * 16, 16)] = jnp.zeros((16,), _f32)

    base = sid * DSTRIPE
    pltpu.sync_copy(z_v, deg_sh.at[pl.ds(base, DSTRIPE)])
    pltpu.sync_copy(eb_ad.at[sid], comb_v)

    plsc.subcore_barrier()

    dsems = (ds0, ds1, ds2, ds3)
    use_hi = cid == 0

    def body(g, carry):
        for b in range(4):
            w = 4 * g + b

            @pl.when(g > 0)
            def _():
                pltpu.make_async_copy(ones_v, deg_sh.at[sidx_v.at[b]],
                                      dsems[b]).wait()

            for c in CHUNKS:
                v = comb_v[w, pl.ds(c, 16)]
                hi = lax.shift_right_logical(v, 16)
                lo = lax.bitwise_and(v, jnp.int32(0xFFFF))
                sidx_v[b, pl.ds(c, 16)] = jnp.where(use_hi, hi, lo)
            pltpu.async_copy(ones_v, deg_sh.at[sidx_v.at[b]], dsems[b],
                             add=True)
        return carry

    lax.fori_loop(0, NWIN // 4, body, 0)
    for b in range(4):
        pltpu.make_async_copy(ones_v, deg_sh.at[sidx_v.at[b]], dsems[b]).wait()

    plsc.subcore_barrier()

    pltpu.sync_copy(deg_sh.at[pl.ds(base, DSTRIPE)], z_v)
    pltpu.sync_copy(z_v, degs.at[pl.ds(cid * DEG_ROWS + base, DSTRIPE)])


@functools.partial(
    pl.kernel,
    out_type=jax.ShapeDtypeStruct((NPAD * NPAD,), _f32),
    mesh=_mesh,
    scratch_types=[
        pltpu.VMEM((NWIN, WIN), _i32),
        pltpu.VMEM((4, WIN), _i32),
        pltpu.VMEM((WIN,), _f32),
        pltpu.VMEM((4 * NPAD,), _f32),
        pltpu.VMEM_SHARED((ACELLS,), _f32),
        pltpu.SemaphoreType.DMA,
        pltpu.SemaphoreType.DMA,
        pltpu.SemaphoreType.DMA,
        pltpu.SemaphoreType.DMA,
    ],
)
def _sc_build_a(eb_ad, a_out, comb_v, fidx, ones_v, zrow, chunk,
                cs0, cs1, cs2, cs3):
    cid = lax.axis_index("c")
    sid = lax.axis_index("s")

    for c in CHUNKS:
        ones_v[pl.ds(c, 16)] = jnp.ones((16,), _f32)

    def zbody(i, carry):
        zrow[pl.ds(16 * i, 16)] = jnp.zeros((16,), _f32)
        return carry

    lax.fori_loop(0, 4 * NPAD // 16, zbody, 0)

    pltpu.sync_copy(eb_ad.at[sid], comb_v)

    csems = (cs0, cs1, cs2, cs3)
    rows_per_tile = ACH // NT

    def pass_body(p, carry0):
        gbase = cid * (NPAD // 2) + p * ACH
        for k in range(4):
            pltpu.sync_copy(
                zrow, chunk.at[pl.ds((sid * rows_per_tile + 4 * k) * NPAD,
                                     4 * NPAD)])
        plsc.subcore_barrier()

        def wbody(g, carry):
            for b in range(4):
                w = 4 * g + b

                @pl.when(g > 0)
                def _():
                    pltpu.make_async_copy(ones_v, chunk.at[fidx.at[b]],
                                          csems[b]).wait()

                for c in CHUNKS:
                    v = comb_v[w, pl.ds(c, 16)]
                    srcv = lax.bitwise_and(v, jnp.int32(0xFFFF))
                    dstv = lax.shift_right_logical(v, 16)
                    rel = dstv - gbase
                    ok = jnp.logical_and(rel >= 0, rel < ACH)
                    rel = jnp.where(ok, rel, ACH)
                    fidx[b, pl.ds(c, 16)] = rel * NPAD + srcv
                pltpu.async_copy(ones_v, chunk.at[fidx.at[b]], csems[b],
                                 add=True)
            return carry

        lax.fori_loop(0, NWIN // 4, wbody, 0)
        for b in range(4):
            pltpu.make_async_copy(ones_v, chunk.at[fidx.at[b]],
                                  csems[b]).wait()
        plsc.subcore_barrier()

        pltpu.sync_copy(
            chunk.at[pl.ds(sid * rows_per_tile * NPAD, rows_per_tile * NPAD)],
            a_out.at[pl.ds((gbase + sid * rows_per_tile) * NPAD,
                           rows_per_tile * NPAD)])
        plsc.subcore_barrier()
        return carry0

    lax.fori_loop(0, NPASS, pass_body, 0)


_bf16 = jnp.bfloat16


def _agg_block(ab, ya_ref, yp_ref, aggp_ref, agga_ref, i):
    aggp_ref[...] = jnp.dot(ab, ya_ref[...].astype(_bf16),
                            preferred_element_type=_f32)
    contrib = lax.dot_general(ab, yp_ref[...].astype(_bf16),
                              (((0,), (0,)), ((), ())),
                              preferred_element_type=_f32)

    @pl.when(i == 0)
    def _():
        agga_ref[...] = contrib

    @pl.when(i > 0)
    def _():
        agga_ref[...] = agga_ref[...] + contrib


def _tc_agg1_body(a_ref, ya_ref, yp_ref, aggp_ref, agga_ref, abf_ref):
    ab = a_ref[...].astype(_bf16)
    abf_ref[...] = ab
    _agg_block(ab, ya_ref, yp_ref, aggp_ref, agga_ref, pl.program_id(0))


def _tc_agg2_body(a_ref, ya_ref, yp_ref, aggp_ref, agga_ref):
    _agg_block(a_ref[...], ya_ref, yp_ref, aggp_ref, agga_ref,
               pl.program_id(0))


_AGG_IN_SPECS = [
    pl.BlockSpec((ACH, NPAD), lambda i: (i, 0)),
    pl.BlockSpec((NPAD, D), lambda i: (0, 0)),
    pl.BlockSpec((ACH, D), lambda i: (i, 0)),
]
_AGG_OUT_SPECS = [
    pl.BlockSpec((ACH, D), lambda i: (i, 0)),
    pl.BlockSpec((NPAD, D), lambda i: (0, 0)),
]
_AGG_OUT_SHAPES = (
    jax.ShapeDtypeStruct((NPAD, D), _f32),
    jax.ShapeDtypeStruct((NPAD, D), _f32),
)

_tc_agg1 = pl.pallas_call(
    _tc_agg1_body,
    grid=(NPAD // ACH,),
    in_specs=_AGG_IN_SPECS,
    out_specs=_AGG_OUT_SPECS + [pl.BlockSpec((ACH, NPAD), lambda i: (i, 0))],
    out_shape=_AGG_OUT_SHAPES + (jax.ShapeDtypeStruct((NPAD, NPAD), _bf16),),
)

_tc_agg2 = pl.pallas_call(
    _tc_agg2_body,
    grid=(NPAD // ACH,),
    in_specs=_AGG_IN_SPECS,
    out_specs=_AGG_OUT_SPECS,
    out_shape=_AGG_OUT_SHAPES,
)


def _tc_pre_body(xa_ref, da_ref, w1_ref, xp_ref, dp_ref, w2_ref, y_ref):
    rs_a = lax.rsqrt(jnp.maximum(da_ref[...], 1.0))
    rs_p = lax.rsqrt(jnp.maximum(dp_ref[...], 1.0))
    y_ref[0] = jnp.dot(xa_ref[...] * rs_a, w1_ref[...],
                       preferred_element_type=_f32)
    y_ref[1] = jnp.dot(xp_ref[...] * rs_p, w2_ref[...],
                       preferred_element_type=_f32)


_tc_pre = pl.pallas_call(
    _tc_pre_body,
    out_shape=jax.ShapeDtypeStruct((2, NPAD, D), _f32),
)


def _tc_mid_body(agga_ref, da_ref, ba_ref, wa_ref,
                 aggp_ref, dp_ref, bp_ref, wp_ref, y_ref):
    rs_a = lax.rsqrt(jnp.maximum(da_ref[...], 1.0))
    rs_p = lax.rsqrt(jnp.maximum(dp_ref[...], 1.0))
    h_a = jax.nn.relu(agga_ref[...] * rs_a + ba_ref[...])
    h_p = jax.nn.relu(aggp_ref[...] * rs_p + bp_ref[...])
    y_ref[0] = jnp.dot(h_a * rs_a, wa_ref[...],
                       preferred_element_type=_f32)
    y_ref[1] = jnp.dot(h_p * rs_p, wp_ref[...],
                       preferred_element_type=_f32)


_tc_mid = pl.pallas_call(
    _tc_mid_body,
    out_shape=jax.ShapeDtypeStruct((2, NPAD, D), _f32),
)


def _tc_out_body(agga_ref, da_ref, ba_ref, aggp_ref, dp_ref, bp_ref,
                 oa_ref, op_ref):
    rs_a = lax.rsqrt(jnp.maximum(da_ref[...], 1.0))
    rs_p = lax.rsqrt(jnp.maximum(dp_ref[...], 1.0))
    oa_ref[...] = agga_ref[...] * rs_a + ba_ref[...]
    op_ref[...] = aggp_ref[...] * rs_p + bp_ref[...]


_tc_out = pl.pallas_call(
    _tc_out_body,
    out_shape=(
        jax.ShapeDtypeStruct((NPAD, D), _f32),
        jax.ShapeDtypeStruct((NPAD, D), _f32),
    ),
)


def kernel(x_author, x_paper, edge_src, edge_dst,
           W1w, b1w, W1b, b1b, W2w, b2w, W2b, b2b, W3w, b3w, W3b, b3b):
    src = edge_src.astype(_i32).reshape(NT, NWIN, WIN)
    dst = edge_dst.astype(_i32).reshape(NT, NWIN, WIN)
    eb_ad = src + (dst << 16)

    degs = _sc_degrees(eb_ad)
    deg_p = degs[:DEG_ROWS].reshape(NPAD, 1)
    deg_a = degs[DEG_ROWS:].reshape(NPAD, 1)

    a_mat = _sc_build_a(eb_ad).reshape(NPAD, NPAD)

    xa = jnp.pad(x_author, ((0, NPAD - N_A), (0, 0)))
    xp = jnp.pad(x_paper, ((0, NPAD - N_P), (0, 0)))

    b1w_ = b1w.reshape(1, D)
    b1b_ = b1b.reshape(1, D)
    b2w_ = b2w.reshape(1, D)
    b2b_ = b2b.reshape(1, D)
    b3w_ = b3w.reshape(1, D)
    b3b_ = b3b.reshape(1, D)

    y1 = _tc_pre(xa, deg_a, W1w, xp, deg_p, W1b)
    agg_p1, agg_a1, a_bf = _tc_agg1(a_mat, y1[0], y1[1])

    y2 = _tc_mid(agg_a1, deg_a, b1b_, W2w, agg_p1, deg_p, b1w_, W2b)
    agg_p2, agg_a2 = _tc_agg2(a_bf, y2[0], y2[1])

    y3 = _tc_mid(agg_a2, deg_a, b2b_, W3w, agg_p2, deg_p, b2w_, W3b)
    agg_p3, agg_a3 = _tc_agg2(a_bf, y3[0], y3[1])

    out_a, out_p = _tc_out(agg_a3, deg_a, b3b_, agg_p3, deg_p, b3w_)
    return (out_a[:N_A], out_p[:N_P])


# speedup vs baseline: 11.4443x; 1.0391x over previous
import functools

import jax
import jax.numpy as jnp
from jax import lax
from jax.experimental import pallas as pl
from jax.experimental.pallas import tpu as pltpu
from jax.experimental.pallas import tpu_sc as plsc

N_A = 5000
N_P = 5000
E_TOT = 320000
D = 128

NPAD = 5120
NT = 16
WIN = 100
EPT = E_TOT // NT
NWIN = EPT // WIN
CHUNKS = list(range(0, WIN - 15, 16)) + ([WIN - 16] if WIN % 16 else [])

DEG_ROWS = NPAD
DSTRIPE = DEG_ROWS // NT

ACH = 256
NPASS = (NPAD // 2) // ACH
ACELLS = (ACH + 1) * NPAD

_f32 = jnp.float32
_i32 = jnp.int32

_mesh = plsc.VectorSubcoreMesh(core_axis_name="c", subcore_axis_name="s")


@functools.partial(
    pl.kernel,
    out_type=jax.ShapeDtypeStruct((2 * DEG_ROWS,), _f32),
    mesh=_mesh,
    scratch_types=[
        pltpu.VMEM((NWIN, WIN), _i32),
        pltpu.VMEM((4, WIN), _i32),
        pltpu.VMEM((WIN,), _f32),
        pltpu.VMEM((DSTRIPE,), _f32),
        pltpu.VMEM_SHARED((DEG_ROWS,), _f32),
        pltpu.SemaphoreType.DMA,
        pltpu.SemaphoreType.DMA,
        pltpu.SemaphoreType.DMA,
        pltpu.SemaphoreType.DMA,
    ],
)
def _sc_degrees(eb_ad, degs, comb_v, sidx_v, ones_v, z_v, deg_sh,
                ds0, ds1, ds2, ds3):
    cid = lax.axis_index("c")
    sid = lax.axis_index("s")

    for c in CHUNKS:
        ones_v[pl.ds(c, 16)] = jnp.ones((16,), _f32)
    for k in range(DSTRIPE // 16):
        z_v[pl.ds(k * 16, 16)] = jnp.zeros((16,), _f32)

    base = sid * DSTRIPE
    pltpu.sync_copy(z_v, deg_sh.at[pl.ds(base, DSTRIPE)])
    pltpu.sync_copy(eb_ad.at[sid], comb_v)

    plsc.subcore_barrier()

    dsems = (ds0, ds1, ds2, ds3)
    use_hi = cid == 0

    def body(g, carry):
        for b in range(4):
            w = 4 * g + b

            @pl.when(g > 0)
            def _():
                pltpu.make_async_copy(ones_v, deg_sh.at[sidx_v.at[b]],
                                      dsems[b]).wait()

            for c in CHUNKS:
                v = comb_v[w, pl.ds(c, 16)]
                hi = lax.shift_right_logical(v, 16)
                lo = lax.bitwise_and(v, jnp.int32(0xFFFF))
                sidx_v[b, pl.ds(c, 16)] = jnp.where(use_hi, hi, lo)
            pltpu.async_copy(ones_v, deg_sh.at[sidx_v.at[b]], dsems[b],
                             add=True)
        return carry

    lax.fori_loop(0, NWIN // 4, body, 0)
    for b in range(4):
        pltpu.make_async_copy(ones_v, deg_sh.at[sidx_v.at[b]], dsems[b]).wait()

    plsc.subcore_barrier()

    pltpu.sync_copy(deg_sh.at[pl.ds(base, DSTRIPE)], z_v)
    pltpu.sync_copy(z_v, degs.at[pl.ds(cid * DEG_ROWS + base, DSTRIPE)])


@functools.partial(
    pl.kernel,
    out_type=jax.ShapeDtypeStruct((NPAD * NPAD,), _f32),
    mesh=_mesh,
    scratch_types=[
        pltpu.VMEM((NWIN, WIN), _i32),
        pltpu.VMEM((4, WIN), _i32),
        pltpu.VMEM((WIN,), _f32),
        pltpu.VMEM((4 * NPAD,), _f32),
        pltpu.VMEM_SHARED((ACELLS,), _f32),
        pltpu.SemaphoreType.DMA,
        pltpu.SemaphoreType.DMA,
        pltpu.SemaphoreType.DMA,
        pltpu.SemaphoreType.DMA,
    ],
)
def _sc_build_a(eb_ad, a_out, comb_v, fidx, ones_v, zrow, chunk,
                cs0, cs1, cs2, cs3):
    cid = lax.axis_index("c")
    sid = lax.axis_index("s")

    for c in CHUNKS:
        ones_v[pl.ds(c, 16)] = jnp.ones((16,), _f32)

    def zbody(i, carry):
        zrow[pl.ds(16 * i, 16)] = jnp.zeros((16,), _f32)
        return carry

    lax.fori_loop(0, 4 * NPAD // 16, zbody, 0)

    pltpu.sync_copy(eb_ad.at[sid], comb_v)

    csems = (cs0, cs1, cs2, cs3)
    rows_per_tile = ACH // NT

    def pass_body(p, carry0):
        gbase = cid * (NPAD // 2) + p * ACH
        for k in range(4):
            pltpu.sync_copy(
                zrow, chunk.at[pl.ds((sid * rows_per_tile + 4 * k) * NPAD,
                                     4 * NPAD)])
        plsc.subcore_barrier()

        def wbody(g, carry):
            for b in range(4):
                w = 4 * g + b

                @pl.when(g > 0)
                def _():
                    pltpu.make_async_copy(ones_v, chunk.at[fidx.at[b]],
                                          csems[b]).wait()

                for c in CHUNKS:
                    v = comb_v[w, pl.ds(c, 16)]
                    srcv = lax.bitwise_and(v, jnp.int32(0xFFFF))
                    dstv = lax.shift_right_logical(v, 16)
                    rel = dstv - gbase
                    ok = jnp.logical_and(rel >= 0, rel < ACH)
                    rel = jnp.where(ok, rel, ACH)
                    fidx[b, pl.ds(c, 16)] = rel * NPAD + srcv
                pltpu.async_copy(ones_v, chunk.at[fidx.at[b]], csems[b],
                                 add=True)
            return carry

        lax.fori_loop(0, NWIN // 4, wbody, 0)
        for b in range(4):
            pltpu.make_async_copy(ones_v, chunk.at[fidx.at[b]],
                                  csems[b]).wait()
        plsc.subcore_barrier()

        pltpu.sync_copy(
            chunk.at[pl.ds(sid * rows_per_tile * NPAD, rows_per_tile * NPAD)],
            a_out.at[pl.ds((gbase + sid * rows_per_tile) * NPAD,
                           rows_per_tile * NPAD)])
        plsc.subcore_barrier()
        return carry0

    lax.fori_loop(0, NPASS, pass_body, 0)


_GRID = NPAD // ACH


def _rs(d_ref):
    return lax.rsqrt(jnp.maximum(d_ref[...], 1.0))


def _agg_step(a_ref, ya, yp, aggp_s, agga_s):
    i = pl.program_id(0)
    ab = a_ref[...]
    aggp_s[pl.ds(i * ACH, ACH), :] = jnp.dot(ab, ya,
                                             preferred_element_type=_f32)
    contrib = lax.dot_general(ab, yp[pl.ds(i * ACH, ACH), :],
                              (((0,), (0,)), ((), ())),
                              preferred_element_type=_f32)

    @pl.when(i == 0)
    def _():
        agga_s[...] = contrib

    @pl.when(i > 0)
    def _():
        agga_s[...] = agga_s[...] + contrib


def _mid_epilogue(aggp_s, agga_s, da_ref, dp_ref, ba_ref, wa_ref,
                  bp_ref, wp_ref, yo_ref):
    rs_a = _rs(da_ref)
    rs_p = _rs(dp_ref)
    h_a = jax.nn.relu(agga_s[...] * rs_a + ba_ref[...])
    h_p = jax.nn.relu(aggp_s[...] * rs_p + bp_ref[...])
    yo_ref[0] = jnp.dot(h_a * rs_a, wa_ref[...], preferred_element_type=_f32)
    yo_ref[1] = jnp.dot(h_p * rs_p, wp_ref[...], preferred_element_type=_f32)


def _tc_l1_body(a_ref, xa_ref, xp_ref, da_ref, dp_ref, w1a_ref, w1p_ref,
                ba_ref, wa_ref, bp_ref, wp_ref, yo_ref,
                y_s, aggp_s, agga_s):
    @pl.when(pl.program_id(0) == 0)
    def _():
        y_s[0] = jnp.dot(xa_ref[...] * _rs(da_ref), w1a_ref[...],
                         preferred_element_type=_f32)
        y_s[1] = jnp.dot(xp_ref[...] * _rs(dp_ref), w1p_ref[...],
                         preferred_element_type=_f32)

    _agg_step(a_ref, y_s[0], y_s.at[1], aggp_s, agga_s)

    @pl.when(pl.program_id(0) == _GRID - 1)
    def _():
        _mid_epilogue(aggp_s, agga_s, da_ref, dp_ref, ba_ref, wa_ref,
                      bp_ref, wp_ref, yo_ref)


def _tc_l2_body(a_ref, y_ref, da_ref, dp_ref, ba_ref, wa_ref,
                bp_ref, wp_ref, yo_ref, aggp_s, agga_s):
    _agg_step(a_ref, y_ref[0], y_ref.at[1], aggp_s, agga_s)

    @pl.when(pl.program_id(0) == _GRID - 1)
    def _():
        _mid_epilogue(aggp_s, agga_s, da_ref, dp_ref, ba_ref, wa_ref,
                      bp_ref, wp_ref, yo_ref)


def _tc_l3_body(a_ref, y_ref, da_ref, dp_ref, ba_ref, bp_ref,
                oa_ref, op_ref, aggp_s, agga_s):
    _agg_step(a_ref, y_ref[0], y_ref.at[1], aggp_s, agga_s)

    @pl.when(pl.program_id(0) == _GRID - 1)
    def _():
        oa_ref[...] = agga_s[...] * _rs(da_ref) + ba_ref[...]
        op_ref[...] = aggp_s[...] * _rs(dp_ref) + bp_ref[...]


def _full(shape):
    nd = len(shape)
    return pl.BlockSpec(shape, lambda i, _n=nd: (0,) * _n)


_A_SPEC = pl.BlockSpec((ACH, NPAD), lambda i: (i, 0))
_Y_SPEC = _full((2, NPAD, D))
_D_SPEC = _full((NPAD, 1))
_B_SPEC = _full((1, D))
_W_SPEC = _full((D, D))
_X_SPEC = _full((NPAD, D))

_tc_l1 = pl.pallas_call(
    _tc_l1_body,
    grid=(_GRID,),
    in_specs=[_A_SPEC, _X_SPEC, _X_SPEC, _D_SPEC, _D_SPEC, _W_SPEC, _W_SPEC,
              _B_SPEC, _W_SPEC, _B_SPEC, _W_SPEC],
    out_specs=_Y_SPEC,
    out_shape=jax.ShapeDtypeStruct((2, NPAD, D), _f32),
    scratch_shapes=[pltpu.VMEM((2, NPAD, D), _f32),
                    pltpu.VMEM((NPAD, D), _f32),
                    pltpu.VMEM((NPAD, D), _f32)],
)

_tc_l2 = pl.pallas_call(
    _tc_l2_body,
    grid=(_GRID,),
    in_specs=[_A_SPEC, _Y_SPEC, _D_SPEC, _D_SPEC,
              _B_SPEC, _W_SPEC, _B_SPEC, _W_SPEC],
    out_specs=_Y_SPEC,
    out_shape=jax.ShapeDtypeStruct((2, NPAD, D), _f32),
    scratch_shapes=[pltpu.VMEM((NPAD, D), _f32),
                    pltpu.VMEM((NPAD, D), _f32)],
)

_tc_l3 = pl.pallas_call(
    _tc_l3_body,
    grid=(_GRID,),
    in_specs=[_A_SPEC, _Y_SPEC, _D_SPEC, _D_SPEC, _B_SPEC, _B_SPEC],
    out_specs=[_full((NPAD, D)), _full((NPAD, D))],
    out_shape=(jax.ShapeDtypeStruct((NPAD, D), _f32),
               jax.ShapeDtypeStruct((NPAD, D), _f32)),
    scratch_shapes=[pltpu.VMEM((NPAD, D), _f32),
                    pltpu.VMEM((NPAD, D), _f32)],
)


def kernel(x_author, x_paper, edge_src, edge_dst,
           W1w, b1w, W1b, b1b, W2w, b2w, W2b, b2b, W3w, b3w, W3b, b3b):
    src = edge_src.astype(_i32).reshape(NT, NWIN, WIN)
    dst = edge_dst.astype(_i32).reshape(NT, NWIN, WIN)
    eb_ad = src + (dst << 16)

    degs = _sc_degrees(eb_ad)
    deg_p = degs[:DEG_ROWS].reshape(NPAD, 1)
    deg_a = degs[DEG_ROWS:].reshape(NPAD, 1)

    a_mat = _sc_build_a(eb_ad).reshape(NPAD, NPAD)

    xa = jnp.pad(x_author, ((0, NPAD - N_A), (0, 0)))
    xp = jnp.pad(x_paper, ((0, NPAD - N_P), (0, 0)))

    b1w_ = b1w.reshape(1, D)
    b1b_ = b1b.reshape(1, D)
    b2w_ = b2w.reshape(1, D)
    b2b_ = b2b.reshape(1, D)
    b3w_ = b3w.reshape(1, D)
    b3b_ = b3b.reshape(1, D)

    y2 = _tc_l1(a_mat, xa, xp, deg_a, deg_p, W1w, W1b,
                b1b_, W2w, b1w_, W2b)
    y3 = _tc_l2(a_mat, y2, deg_a, deg_p, b2b_, W3w, b2w_, W3b)
    out_a, out_p = _tc_l3(a_mat, y3, deg_a, deg_p, b3b_, b3w_)
    return (out_a[:N_A], out_p[:N_P])
